# baseline (speedup 1.0000x reference)
import dataclasses
import functools

import jax
import jax.numpy as jnp
import numpy as np
from jax import lax
from jax.experimental import pallas as pl
from jax.experimental.pallas import tpu as pltpu
from jax.experimental.pallas import tpu_sc as plsc

N = 10000
E = 320000
D = 128
DW = D // 2

NC = 2
NS = 16
NW = NC * NS

EPW = E // NW
K = 80
NCHUNK = EPW // K
ZR = 80
IBLK = 25
NIB = NCHUNK // IBLK

_PERM = np.concatenate(
    [np.concatenate([32 * q + 2 * np.arange(16),
                     32 * q + 2 * np.arange(16) + 1]) for q in range(4)]
)

_mesh = plsc.VectorSubcoreMesh(core_axis_name="c", subcore_axis_name="s")

_sc_params = pltpu.CompilerParams()
for _f, _v in (("needs_layout_passes", False),
               ("use_tc_tiling_on_sc", False)):
    if _f in pltpu.CompilerParams.__dataclass_fields__:
        _sc_params = dataclasses.replace(_sc_params, **{_f: _v})


@functools.partial(
    pl.kernel,
    mesh=_mesh,
    compiler_params=_sc_params,
    out_type=[
        jax.ShapeDtypeStruct((N, D), jnp.float32),
        jax.ShapeDtypeStruct((N, D), jnp.float32),
    ],
    scratch_types=[
        pltpu.VMEM_SHARED((N, D), jnp.float32),
        pltpu.VMEM((IBLK, K), jnp.int32),
        pltpu.VMEM((IBLK, K), jnp.int32),
        pltpu.VMEM((IBLK, K), jnp.int32),
        pltpu.VMEM((IBLK, K), jnp.int32),
        pltpu.VMEM((K, DW), jnp.int32),
        pltpu.VMEM((K, DW), jnp.int32),
        pltpu.VMEM((K, D), jnp.float32),
        pltpu.VMEM((K, D), jnp.float32),
        pltpu.SemaphoreType.DMA,
        pltpu.SemaphoreType.DMA,
        pltpu.SemaphoreType.DMA,
        pltpu.SemaphoreType.DMA,
        pltpu.SemaphoreType.DMA,
        pltpu.SemaphoreType.DMA,
        pltpu.SemaphoreType.DMA,
    ],
)
def _segsum(h_hbm, src_hbm, dst_hbm, p0_hbm, p1_hbm,
            acc, srci0, srci1, dsti0, dsti1, g0, g1, f0, f1,
            sg0, sg1, sc0, sc1, si0, si1, zsem):
    cid = lax.axis_index("c")
    sid = lax.axis_index("s")
    wid = cid * NS + sid
    gbuf = [g0, g1]
    fbuf = [f0, f1]
    gsem = [sg0, sg1]
    ssem = [sc0, sc1]
    srcis = [srci0, srci1]
    dstis = [dsti0, dsti1]
    semis = [si0, si1]
    zbuf = f1

    pltpu.sync_copy(src_hbm.at[wid, 0], srcis[0])
    pltpu.sync_copy(dst_hbm.at[wid, 0], dstis[0])
    pltpu.async_copy(src_hbm.at[wid, 1], srcis[1], semis[1])
    pltpu.async_copy(dst_hbm.at[wid, 1], dstis[1], semis[1])

    for i in range(2):
        pltpu.async_copy(h_hbm.at[srcis[0].at[i]], gbuf[i], gsem[i])

    zeros = jnp.zeros((16,), jnp.float32)

    @pl.loop(0, ZR)
    def _(i):
        @pl.loop(0, D // 16)
        def _(j):
            zbuf[i, pl.ds(j * 16, 16)] = zeros

    @pl.loop(sid, N // ZR, step=NS)
    def _(r):
        pltpu.async_copy(zbuf, acc.at[pl.ds(r * ZR, ZR)], zsem)

    @pl.loop(sid, N // ZR, step=NS)
    def _(r):
        pltpu.make_async_copy(zbuf, acc.at[pl.ds(r * ZR, ZR)], zsem).wait()

    plsc.subcore_barrier()

    def unpack_chunk(gb, fb):
        @pl.loop(0, K, step=2)
        def _(i0):
            for di in range(2):
                i = i0 + di
                for q in range(4):
                    w = gb[i, pl.ds(q * 16, 16)]
                    v = plsc.bitcast(w, jnp.bfloat16)
                    a, c = plsc.unpack(v, format=plsc.PackFormat.INTERLEAVED)
                    fb[i, pl.ds(q * 32, 16)] = a
                    fb[i, pl.ds(q * 32 + 16, 16)] = c

    def do_block(sI, dI, nxt_sI):
        @pl.loop(0, IBLK - 1, step=2)
        def _(j):
            for b in range(2):
                lc = j + b
                pltpu.make_async_copy(
                    h_hbm.at[sI.at[lc]], gbuf[b], gsem[b]).wait()

                @pl.when(lc >= 2)
                def _():
                    pltpu.make_async_copy(
                        fbuf[b], acc.at[dI.at[lc - 2]], ssem[b]).wait()

                unpack_chunk(gbuf[b], fbuf[b])

                @pl.when(lc + 2 < IBLK)
                def _():
                    pltpu.async_copy(
                        h_hbm.at[sI.at[lc + 2]], gbuf[b], gsem[b])

                pltpu.async_copy(fbuf[b], acc.at[dI.at[lc]], ssem[b],
                                 add=True)

        t = IBLK - 1
        pltpu.make_async_copy(h_hbm.at[sI.at[t]], gbuf[0], gsem[0]).wait()
        pltpu.make_async_copy(fbuf[0], acc.at[dI.at[t - 2]], ssem[0]).wait()
        unpack_chunk(gbuf[0], fbuf[0])
        if nxt_sI is not None:
            for i in range(2):
                pltpu.async_copy(h_hbm.at[nxt_sI.at[i]], gbuf[i], gsem[i])
        pltpu.async_copy(fbuf[0], acc.at[dI.at[t]], ssem[0], add=True)
        pltpu.make_async_copy(fbuf[1], acc.at[dI.at[t - 1]], ssem[1]).wait()
        pltpu.make_async_copy(fbuf[0], acc.at[dI.at[t]], ssem[0]).wait()

    do_block(srcis[0], dstis[0], srcis[1])
    pltpu.async_copy(src_hbm.at[wid, 2], srcis[0], semis[0])
    pltpu.async_copy(dst_hbm.at[wid, 2], dstis[0], semis[0])

    @pl.loop(1, NIB, step=2)
    def _(ob):
        for dp in range(2):
            ib = ob + dp
            p = (1 + dp) % 2

            @pl.when(ib < NIB)
            def _():
                pltpu.make_async_copy(
                    src_hbm.at[wid, ib], srcis[p], semis[p]).wait()
                pltpu.make_async_copy(
                    dst_hbm.at[wid, ib], dstis[p], semis[p]).wait()

                @pl.when(ib + 1 < NIB)
                def _():
                    do_block(srcis[p], dstis[p], srcis[1 - p])

                @pl.when(ib + 1 >= NIB)
                def _():
                    do_block(srcis[p], dstis[p], None)

                @pl.when(ib + 2 < NIB)
                def _():
                    pltpu.async_copy(
                        src_hbm.at[wid, ib + 2], srcis[p], semis[p])
                    pltpu.async_copy(
                        dst_hbm.at[wid, ib + 2], dstis[p], semis[p])

    plsc.subcore_barrier()

    out_hbm = [p0_hbm, p1_hbm]
    for c in range(NC):
        @pl.when(cid == c)
        def _():
            @pl.loop(sid, N // ZR, step=NS)
            def _(r):
                ds = pl.ds(r * ZR, ZR)
                pltpu.async_copy(acc.at[ds], out_hbm[c].at[ds], zsem)

            @pl.loop(sid, N // ZR, step=NS)
            def _(r):
                ds = pl.ds(r * ZR, ZR)
                pltpu.make_async_copy(acc.at[ds], out_hbm[c].at[ds],
                                      zsem).wait()


_BLK = 1000


def _combine_body(do_relu, p0_ref, p1_ref, h_ref, wrel_ref, b_ref, wroot_ref,
                  o_ref, ob_ref):
    s = p0_ref[...] + p1_ref[...]
    acc = lax.dot_general(
        s, wrel_ref[...], (((1,), (0,)), ((), ())),
        precision=lax.Precision.HIGHEST, preferred_element_type=jnp.float32)
    acc = acc + lax.dot_general(
        h_ref[...], wroot_ref[...], (((1,), (0,)), ((), ())),
        precision=lax.Precision.HIGHEST, preferred_element_type=jnp.float32)
    acc = acc + b_ref[...]
    if do_relu:
        acc = jnp.maximum(acc, 0.0)
    o_ref[...] = acc
    ob_ref[...] = acc.astype(jnp.bfloat16)


def _combine(p0, p1, h, w_rel_p, b_rel, w_root, do_relu):
    return pl.pallas_call(
        functools.partial(_combine_body, do_relu),
        grid=(N // _BLK,),
        in_specs=[
            pl.BlockSpec((_BLK, D), lambda i: (i, 0)),
            pl.BlockSpec((_BLK, D), lambda i: (i, 0)),
            pl.BlockSpec((_BLK, D), lambda i: (i, 0)),
            pl.BlockSpec((D, D), lambda i: (0, 0)),
            pl.BlockSpec((1, D), lambda i: (0, 0)),
            pl.BlockSpec((D, D), lambda i: (0, 0)),
        ],
        out_specs=[pl.BlockSpec((_BLK, D), lambda i: (i, 0)),
                   pl.BlockSpec((_BLK, D), lambda i: (i, 0))],
        out_shape=[jax.ShapeDtypeStruct((N, D), jnp.float32),
                   jax.ShapeDtypeStruct((N, D), jnp.bfloat16)],
    )(p0, p1, h, w_rel_p, b_rel.reshape(1, D), w_root)


def _pack(hb):
    return jax.lax.bitcast_convert_type(hb.reshape(N, DW, 2), jnp.int32)


def kernel(x, edge_index, W_rel0, b_rel0, W_root0, W_rel1, b_rel1, W_root1,
           W_rel2, b_rel2, W_root2):
    src = edge_index[0].reshape(NW, NIB, IBLK, K)
    dst = edge_index[1].reshape(NW, NIB, IBLK, K)
    perm = jnp.asarray(_PERM, dtype=jnp.int32)
    h = x
    hb = x.astype(jnp.bfloat16)
    layers = [
        (W_rel0, b_rel0, W_root0, True),
        (W_rel1, b_rel1, W_root1, True),
        (W_rel2, b_rel2, W_root2, False),
    ]
    for w_rel, b_rel, w_root, do_relu in layers:
        p0, p1 = _segsum(_pack(hb), src, dst)
        w_rel_p = jnp.take(w_rel, perm, axis=0)
        h, hb = _combine(p0, p1, h, w_rel_p, b_rel, w_root, do_relu)
    return h


# speedup vs baseline: 2.0706x; 2.0706x over previous
import dataclasses
import functools

import jax
import jax.numpy as jnp
from jax import lax
from jax.experimental import pallas as pl
from jax.experimental.pallas import tpu as pltpu
from jax.experimental.pallas import tpu_sc as plsc

N = 10000
E = 320000
D = 128

NC = 2
NS = 16
NW = NC * NS

EPW = E // NW
K = 80
NCHUNK = EPW // K
NBUF = 3
ZR = 80
IBLK = 25
NIB = NCHUNK // IBLK

_mesh = plsc.VectorSubcoreMesh(core_axis_name="c", subcore_axis_name="s")

_sc_params = pltpu.CompilerParams()
for _f, _v in (("disable_bounds_checks", True),
               ("disable_semaphore_checks", True)):
    if _f in pltpu.CompilerParams.__dataclass_fields__:
        _sc_params = dataclasses.replace(_sc_params, **{_f: _v})


@functools.partial(
    pl.kernel,
    mesh=_mesh,
    compiler_params=_sc_params,
    out_type=[
        jax.ShapeDtypeStruct((N, D), jnp.float32),
        jax.ShapeDtypeStruct((N, D), jnp.float32),
    ],
    scratch_types=[
        pltpu.VMEM_SHARED((N, D), jnp.float32),
        pltpu.VMEM((IBLK, K), jnp.int32),
        pltpu.VMEM((IBLK, K), jnp.int32),
        pltpu.VMEM((IBLK, K), jnp.int32),
        pltpu.VMEM((IBLK, K), jnp.int32),
        pltpu.VMEM((K, D), jnp.float32),
        pltpu.VMEM((K, D), jnp.float32),
        pltpu.VMEM((K, D), jnp.float32),
        pltpu.SemaphoreType.DMA,
        pltpu.SemaphoreType.DMA,
        pltpu.SemaphoreType.DMA,
        pltpu.SemaphoreType.DMA,
        pltpu.SemaphoreType.DMA,
        pltpu.SemaphoreType.DMA,
        pltpu.SemaphoreType.DMA,
        pltpu.SemaphoreType.DMA,
        pltpu.SemaphoreType.DMA,
    ],
)
def _segsum(h_hbm, src_hbm, dst_hbm, p0_hbm, p1_hbm,
            acc, srci0, srci1, dsti0, dsti1, r0, r1, r2,
            s0, s1, s2, c0, c1, c2, si0, si1, zsem):
    cid = lax.axis_index("c")
    sid = lax.axis_index("s")
    wid = cid * NS + sid
    rows = [r0, r1, r2]
    sems = [s0, s1, s2]
    ssems = [c0, c1, c2]
    srcis = [srci0, srci1]
    dstis = [dsti0, dsti1]
    semis = [si0, si1]
    zbuf = r2

    pltpu.sync_copy(src_hbm.at[wid, 0], srcis[0])
    pltpu.sync_copy(dst_hbm.at[wid, 0], dstis[0])
    pltpu.async_copy(src_hbm.at[wid, 1], srcis[1], semis[1])
    pltpu.async_copy(dst_hbm.at[wid, 1], dstis[1], semis[1])

    for i in range(NBUF - 1):
        pltpu.async_copy(h_hbm.at[srcis[0].at[i]], rows[i], sems[i])

    zeros = jnp.zeros((16,), jnp.float32)

    @pl.loop(0, ZR)
    def _(i):
        @pl.loop(0, D // 16)
        def _(j):
            zbuf[i, pl.ds(j * 16, 16)] = zeros

    @pl.loop(sid, N // ZR, step=NS)
    def _(r):
        pltpu.async_copy(zbuf, acc.at[pl.ds(r * ZR, ZR)], zsem)

    @pl.loop(sid, N // ZR, step=NS)
    def _(r):
        pltpu.make_async_copy(zbuf, acc.at[pl.ds(r * ZR, ZR)], zsem).wait()

    plsc.subcore_barrier()

    def do_block(sI, dI, prime):
        if prime:
            for i in range(NBUF - 1):
                pltpu.async_copy(h_hbm.at[sI.at[i]], rows[i], sems[i])

        @pl.loop(0, IBLK - 1, step=NBUF)
        def _(j):
            for b in range(NBUF):
                lc = j + b
                nf = lc + NBUF - 1
                pf = (NBUF - 1 + b) % NBUF

                @pl.when(nf < IBLK)
                def _():
                    def _wait_prev():
                        pltpu.make_async_copy(
                            rows[pf], acc.at[dI.at[lc - 1]],
                            ssems[pf]).wait()
                    if b == 0:
                        pl.when(lc > 0)(_wait_prev)
                    else:
                        _wait_prev()
                    pltpu.async_copy(h_hbm.at[sI.at[nf]], rows[pf], sems[pf])

                pltpu.make_async_copy(
                    h_hbm.at[sI.at[lc]], rows[b], sems[b]).wait()
                pltpu.async_copy(rows[b], acc.at[dI.at[lc]], ssems[b],
                                 add=True)

        pltpu.make_async_copy(
            h_hbm.at[sI.at[IBLK - 1]], rows[0], sems[0]).wait()
        pltpu.async_copy(rows[0], acc.at[dI.at[IBLK - 1]], ssems[0],
                         add=True)
        for c in range(IBLK - NBUF, IBLK):
            s = c % NBUF
            pltpu.make_async_copy(rows[s], acc.at[dI.at[c]], ssems[s]).wait()

    do_block(srcis[0], dstis[0], prime=False)
    pltpu.async_copy(src_hbm.at[wid, 2], srcis[0], semis[0])
    pltpu.async_copy(dst_hbm.at[wid, 2], dstis[0], semis[0])

    @pl.loop(1, NIB, step=2)
    def _(ob):
        for dp in range(2):
            ib = ob + dp
            p = (1 + dp) % 2

            @pl.when(ib < NIB)
            def _():
                pltpu.make_async_copy(
                    src_hbm.at[wid, ib], srcis[p], semis[p]).wait()
                pltpu.make_async_copy(
                    dst_hbm.at[wid, ib], dstis[p], semis[p]).wait()

                do_block(srcis[p], dstis[p], prime=True)

                @pl.when(ib + 2 < NIB)
                def _():
                    pltpu.async_copy(
                        src_hbm.at[wid, ib + 2], srcis[p], semis[p])
                    pltpu.async_copy(
                        dst_hbm.at[wid, ib + 2], dstis[p], semis[p])

    plsc.subcore_barrier()

    out_hbm = [p0_hbm, p1_hbm]
    for c in range(NC):
        @pl.when(cid == c)
        def _():
            @pl.loop(sid, N // ZR, step=NS)
            def _(r):
                ds = pl.ds(r * ZR, ZR)
                pltpu.async_copy(acc.at[ds], out_hbm[c].at[ds], zsem)

            @pl.loop(sid, N // ZR, step=NS)
            def _(r):
                ds = pl.ds(r * ZR, ZR)
                pltpu.make_async_copy(acc.at[ds], out_hbm[c].at[ds],
                                      zsem).wait()


_BLK = 1000


def _combine_body(do_relu, p0_ref, p1_ref, h_ref, wrel_ref, b_ref, wroot_ref,
                  o_ref):
    s = p0_ref[...] + p1_ref[...]
    acc = lax.dot_general(
        s, wrel_ref[...], (((1,), (0,)), ((), ())),
        precision=lax.Precision.HIGHEST, preferred_element_type=jnp.float32)
    acc = acc + lax.dot_general(
        h_ref[...], wroot_ref[...], (((1,), (0,)), ((), ())),
        precision=lax.Precision.HIGHEST, preferred_element_type=jnp.float32)
    acc = acc + b_ref[...]
    if do_relu:
        acc = jnp.maximum(acc, 0.0)
    o_ref[...] = acc


def _combine(p0, p1, h, w_rel, b_rel, w_root, do_relu):
    return pl.pallas_call(
        functools.partial(_combine_body, do_relu),
        grid=(N // _BLK,),
        in_specs=[
            pl.BlockSpec((_BLK, D), lambda i: (i, 0)),
            pl.BlockSpec((_BLK, D), lambda i: (i, 0)),
            pl.BlockSpec((_BLK, D), lambda i: (i, 0)),
            pl.BlockSpec((D, D), lambda i: (0, 0)),
            pl.BlockSpec((1, D), lambda i: (0, 0)),
            pl.BlockSpec((D, D), lambda i: (0, 0)),
        ],
        out_specs=pl.BlockSpec((_BLK, D), lambda i: (i, 0)),
        out_shape=jax.ShapeDtypeStruct((N, D), jnp.float32),
    )(p0, p1, h, w_rel, b_rel.reshape(1, D), w_root)


def kernel(x, edge_index, W_rel0, b_rel0, W_root0, W_rel1, b_rel1, W_root1,
           W_rel2, b_rel2, W_root2):
    src = edge_index[0].reshape(NW, NIB, IBLK, K)
    dst = edge_index[1].reshape(NW, NIB, IBLK, K)
    h = x
    layers = [
        (W_rel0, b_rel0, W_root0, True),
        (W_rel1, b_rel1, W_root1, True),
        (W_rel2, b_rel2, W_root2, False),
    ]
    for w_rel, b_rel, w_root, do_relu in layers:
        p0, p1 = _segsum(h, src, dst)
        h = _combine(p0, p1, h, w_rel, b_rel, w_root, do_relu)
    return h


# speedup vs baseline: 2.0930x; 1.0108x over previous
import dataclasses
import functools

import jax
import jax.numpy as jnp
from jax import lax
from jax.experimental import pallas as pl
from jax.experimental.pallas import tpu as pltpu
from jax.experimental.pallas import tpu_sc as plsc

N = 10000
E = 320000
D = 128

NC = 2
NS = 16
NW = NC * NS

EPW = E // NW
K = 80
NCHUNK = EPW // K
NBUF = 3
ZR = 80
IBLK = 25
NIB = NCHUNK // IBLK

_mesh = plsc.VectorSubcoreMesh(core_axis_name="c", subcore_axis_name="s")

_sc_params = pltpu.CompilerParams()
for _f, _v in (("disable_bounds_checks", True),
               ("disable_semaphore_checks", True)):
    if _f in pltpu.CompilerParams.__dataclass_fields__:
        _sc_params = dataclasses.replace(_sc_params, **{_f: _v})


@functools.partial(
    pl.kernel,
    mesh=_mesh,
    compiler_params=_sc_params,
    out_type=[
        jax.ShapeDtypeStruct((N, D), jnp.float32),
        jax.ShapeDtypeStruct((N, D), jnp.float32),
    ],
    scratch_types=[
        pltpu.VMEM_SHARED((N, D), jnp.float32),
        pltpu.VMEM((IBLK, K), jnp.int32),
        pltpu.VMEM((IBLK, K), jnp.int32),
        pltpu.VMEM((IBLK, K), jnp.int32),
        pltpu.VMEM((IBLK, K), jnp.int32),
        pltpu.VMEM((K, D), jnp.float32),
        pltpu.VMEM((K, D), jnp.float32),
        pltpu.VMEM((K, D), jnp.float32),
        pltpu.SemaphoreType.DMA,
        pltpu.SemaphoreType.DMA,
        pltpu.SemaphoreType.DMA,
        pltpu.SemaphoreType.DMA,
        pltpu.SemaphoreType.DMA,
        pltpu.SemaphoreType.DMA,
        pltpu.SemaphoreType.DMA,
        pltpu.SemaphoreType.DMA,
        pltpu.SemaphoreType.DMA,
    ],
)
def _segsum(h_hbm, src_hbm, dst_hbm, p0_hbm, p1_hbm,
            acc, srci0, srci1, dsti0, dsti1, r0, r1, r2,
            s0, s1, s2, c0, c1, c2, si0, si1, zsem):
    cid = lax.axis_index("c")
    sid = lax.axis_index("s")
    wid = cid * NS + sid
    rows = [r0, r1, r2]
    sems = [s0, s1, s2]
    ssems = [c0, c1, c2]
    srcis = [srci0, srci1]
    dstis = [dsti0, dsti1]
    semis = [si0, si1]
    zbuf = r2

    zeros = jnp.zeros((16,), jnp.float32)

    @pl.loop(0, ZR)
    def _(i):
        @pl.loop(0, D // 16)
        def _(j):
            zbuf[i, pl.ds(j * 16, 16)] = zeros

    @pl.loop(sid, N // ZR, step=NS)
    def _(r):
        pltpu.async_copy(zbuf, acc.at[pl.ds(r * ZR, ZR)], zsem)

    pltpu.async_copy(src_hbm.at[wid, 0], srcis[0], semis[0])
    pltpu.async_copy(dst_hbm.at[wid, 0], dstis[0], semis[0])
    pltpu.async_copy(src_hbm.at[wid, 1], srcis[1], semis[1])
    pltpu.async_copy(dst_hbm.at[wid, 1], dstis[1], semis[1])
    pltpu.make_async_copy(src_hbm.at[wid, 0], srcis[0], semis[0]).wait()
    pltpu.make_async_copy(dst_hbm.at[wid, 0], dstis[0], semis[0]).wait()

    for i in range(NBUF - 1):
        pltpu.async_copy(h_hbm.at[srcis[0].at[i]], rows[i], sems[i])

    @pl.loop(sid, N // ZR, step=NS)
    def _(r):
        pltpu.make_async_copy(zbuf, acc.at[pl.ds(r * ZR, ZR)], zsem).wait()

    plsc.subcore_barrier()

    def do_block(sI, dI, prime):
        if prime:
            for i in range(NBUF - 1):
                pltpu.async_copy(h_hbm.at[sI.at[i]], rows[i], sems[i])

        @pl.loop(0, IBLK - 1, step=NBUF)
        def _(j):
            for b in range(NBUF):
                lc = j + b
                nf = lc + NBUF - 1
                pf = (NBUF - 1 + b) % NBUF

                @pl.when(nf < IBLK)
                def _():
                    def _wait_prev():
                        pltpu.make_async_copy(
                            rows[pf], acc.at[dI.at[lc - 1]],
                            ssems[pf]).wait()
                    if b == 0:
                        pl.when(lc > 0)(_wait_prev)
                    else:
                        _wait_prev()
                    pltpu.async_copy(h_hbm.at[sI.at[nf]], rows[pf], sems[pf])

                pltpu.make_async_copy(
                    h_hbm.at[sI.at[lc]], rows[b], sems[b]).wait()
                pltpu.async_copy(rows[b], acc.at[dI.at[lc]], ssems[b],
                                 add=True)

        pltpu.make_async_copy(
            h_hbm.at[sI.at[IBLK - 1]], rows[0], sems[0]).wait()
        pltpu.async_copy(rows[0], acc.at[dI.at[IBLK - 1]], ssems[0],
                         add=True)
        for c in range(IBLK - NBUF, IBLK):
            s = c % NBUF
            pltpu.make_async_copy(rows[s], acc.at[dI.at[c]], ssems[s]).wait()

    do_block(srcis[0], dstis[0], prime=False)
    pltpu.async_copy(src_hbm.at[wid, 2], srcis[0], semis[0])
    pltpu.async_copy(dst_hbm.at[wid, 2], dstis[0], semis[0])

    @pl.loop(1, NIB, step=2)
    def _(ob):
        for dp in range(2):
            ib = ob + dp
            p = (1 + dp) % 2

            @pl.when(ib < NIB)
            def _():
                pltpu.make_async_copy(
                    src_hbm.at[wid, ib], srcis[p], semis[p]).wait()
                pltpu.make_async_copy(
                    dst_hbm.at[wid, ib], dstis[p], semis[p]).wait()

                do_block(srcis[p], dstis[p], prime=True)

                @pl.when(ib + 2 < NIB)
                def _():
                    pltpu.async_copy(
                        src_hbm.at[wid, ib + 2], srcis[p], semis[p])
                    pltpu.async_copy(
                        dst_hbm.at[wid, ib + 2], dstis[p], semis[p])

    plsc.subcore_barrier()

    out_hbm = [p0_hbm, p1_hbm]
    for c in range(NC):
        @pl.when(cid == c)
        def _():
            @pl.loop(sid, N // ZR, step=NS)
            def _(r):
                ds = pl.ds(r * ZR, ZR)
                pltpu.async_copy(acc.at[ds], out_hbm[c].at[ds], zsem)

            @pl.loop(sid, N // ZR, step=NS)
            def _(r):
                ds = pl.ds(r * ZR, ZR)
                pltpu.make_async_copy(acc.at[ds], out_hbm[c].at[ds],
                                      zsem).wait()


_BLK = 1000


def _combine_body(do_relu, p0_ref, p1_ref, h_ref, wrel_ref, b_ref, wroot_ref,
                  o_ref):
    s = p0_ref[...] + p1_ref[...]
    acc = lax.dot_general(
        s, wrel_ref[...], (((1,), (0,)), ((), ())),
        precision=lax.Precision.HIGHEST, preferred_element_type=jnp.float32)
    acc = acc + lax.dot_general(
        h_ref[...], wroot_ref[...], (((1,), (0,)), ((), ())),
        precision=lax.Precision.HIGHEST, preferred_element_type=jnp.float32)
    acc = acc + b_ref[...]
    if do_relu:
        acc = jnp.maximum(acc, 0.0)
    o_ref[...] = acc


def _combine(p0, p1, h, w_rel, b_rel, w_root, do_relu):
    return pl.pallas_call(
        functools.partial(_combine_body, do_relu),
        grid=(N // _BLK,),
        in_specs=[
            pl.BlockSpec((_BLK, D), lambda i: (i, 0)),
            pl.BlockSpec((_BLK, D), lambda i: (i, 0)),
            pl.BlockSpec((_BLK, D), lambda i: (i, 0)),
            pl.BlockSpec((D, D), lambda i: (0, 0)),
            pl.BlockSpec((1, D), lambda i: (0, 0)),
            pl.BlockSpec((D, D), lambda i: (0, 0)),
        ],
        out_specs=pl.BlockSpec((_BLK, D), lambda i: (i, 0)),
        out_shape=jax.ShapeDtypeStruct((N, D), jnp.float32),
    )(p0, p1, h, w_rel, b_rel.reshape(1, D), w_root)


def kernel(x, edge_index, W_rel0, b_rel0, W_root0, W_rel1, b_rel1, W_root1,
           W_rel2, b_rel2, W_root2):
    src = edge_index[0].reshape(NW, NIB, IBLK, K)
    dst = edge_index[1].reshape(NW, NIB, IBLK, K)
    h = x
    layers = [
        (W_rel0, b_rel0, W_root0, True),
        (W_rel1, b_rel1, W_root1, True),
        (W_rel2, b_rel2, W_root2, False),
    ]
    for w_rel, b_rel, w_root, do_relu in layers:
        p0, p1 = _segsum(h, src, dst)
        h = _combine(p0, p1, h, w_rel, b_rel, w_root, do_relu)
    return h


# speedup vs baseline: 2.1718x; 1.0376x over previous
import dataclasses
import functools

import jax
import jax.numpy as jnp
from jax import lax
from jax.experimental import pallas as pl
from jax.experimental.pallas import tpu as pltpu
from jax.experimental.pallas import tpu_sc as plsc

N = 10000
E = 320000
D = 128

NC = 2
NS = 16
NW = NC * NS

EPW = E // NW
K = 80
NCHUNK = EPW // K
NBUF = 3
ZR = 80
IBLK = 25
NIB = NCHUNK // IBLK

_mesh = plsc.VectorSubcoreMesh(core_axis_name="c", subcore_axis_name="s")

_sc_params = pltpu.CompilerParams()
for _f, _v in (("disable_bounds_checks", True),
               ("disable_semaphore_checks", True)):
    if _f in pltpu.CompilerParams.__dataclass_fields__:
        _sc_params = dataclasses.replace(_sc_params, **{_f: _v})


@functools.partial(
    pl.kernel,
    mesh=_mesh,
    compiler_params=_sc_params,
    out_type=[
        jax.ShapeDtypeStruct((N, D), jnp.float32),
        jax.ShapeDtypeStruct((N, D), jnp.float32),
    ],
    scratch_types=[
        pltpu.VMEM_SHARED((N, D), jnp.float32),
        pltpu.VMEM((IBLK, K), jnp.int32),
        pltpu.VMEM((IBLK, K), jnp.int32),
        pltpu.VMEM((IBLK, K), jnp.int32),
        pltpu.VMEM((IBLK, K), jnp.int32),
        pltpu.VMEM((K, D), jnp.float32),
        pltpu.VMEM((K, D), jnp.float32),
        pltpu.VMEM((K, D), jnp.float32),
        pltpu.SemaphoreType.DMA,
        pltpu.SemaphoreType.DMA,
        pltpu.SemaphoreType.DMA,
        pltpu.SemaphoreType.DMA,
        pltpu.SemaphoreType.DMA,
        pltpu.SemaphoreType.DMA,
        pltpu.SemaphoreType.DMA,
        pltpu.SemaphoreType.DMA,
        pltpu.SemaphoreType.DMA,
    ],
)
def _segsum(h_hbm, src_hbm, dst_hbm, p0_hbm, p1_hbm,
            acc, srci0, srci1, dsti0, dsti1, r0, r1, r2,
            s0, s1, s2, c0, c1, c2, si0, si1, zsem):
    cid = lax.axis_index("c")
    sid = lax.axis_index("s")
    wid = cid * NS + sid
    rows = [r0, r1, r2]
    sems = [s0, s1, s2]
    ssems = [c0, c1, c2]
    srcis = [srci0, srci1]
    dstis = [dsti0, dsti1]
    semis = [si0, si1]
    zbuf = r2

    zeros = jnp.zeros((16,), jnp.float32)

    @pl.loop(0, ZR)
    def _(i):
        @pl.loop(0, D // 16)
        def _(j):
            zbuf[i, pl.ds(j * 16, 16)] = zeros

    @pl.loop(sid, N // ZR, step=NS)
    def _(r):
        pltpu.async_copy(zbuf, acc.at[pl.ds(r * ZR, ZR)], zsem)

    pltpu.async_copy(src_hbm.at[wid, 0], srcis[0], semis[0])
    pltpu.async_copy(dst_hbm.at[wid, 0], dstis[0], semis[0])
    pltpu.async_copy(src_hbm.at[wid, 1], srcis[1], semis[1])
    pltpu.async_copy(dst_hbm.at[wid, 1], dstis[1], semis[1])
    pltpu.make_async_copy(src_hbm.at[wid, 0], srcis[0], semis[0]).wait()
    pltpu.make_async_copy(dst_hbm.at[wid, 0], dstis[0], semis[0]).wait()

    for i in range(NBUF - 1):
        pltpu.async_copy(h_hbm.at[srcis[0].at[i]], rows[i], sems[i])

    @pl.loop(sid, N // ZR, step=NS)
    def _(r):
        pltpu.make_async_copy(zbuf, acc.at[pl.ds(r * ZR, ZR)], zsem).wait()

    plsc.subcore_barrier()

    def do_block(sI, dI, prime):
        if prime:
            for i in range(NBUF - 1):
                pltpu.async_copy(h_hbm.at[sI.at[i]], rows[i], sems[i])

        @pl.loop(0, IBLK - 1, step=NBUF)
        def _(j):
            for b in range(NBUF):
                lc = j + b
                nf = lc + NBUF - 1
                pf = (NBUF - 1 + b) % NBUF

                @pl.when(nf < IBLK)
                def _():
                    def _wait_prev():
                        pltpu.make_async_copy(
                            rows[pf], acc.at[dI.at[lc - 1]],
                            ssems[pf]).wait()
                    if b == 0:
                        pl.when(lc > 0)(_wait_prev)
                    else:
                        _wait_prev()
                    pltpu.async_copy(h_hbm.at[sI.at[nf]], rows[pf], sems[pf])

                pltpu.make_async_copy(
                    h_hbm.at[sI.at[lc]], rows[b], sems[b]).wait()
                pltpu.async_copy(rows[b], acc.at[dI.at[lc]], ssems[b],
                                 add=True)

        pltpu.make_async_copy(
            h_hbm.at[sI.at[IBLK - 1]], rows[0], sems[0]).wait()
        pltpu.async_copy(rows[0], acc.at[dI.at[IBLK - 1]], ssems[0],
                         add=True)
        for c in range(IBLK - NBUF, IBLK):
            s = c % NBUF
            pltpu.make_async_copy(rows[s], acc.at[dI.at[c]], ssems[s]).wait()

    do_block(srcis[0], dstis[0], prime=False)
    pltpu.async_copy(src_hbm.at[wid, 2], srcis[0], semis[0])
    pltpu.async_copy(dst_hbm.at[wid, 2], dstis[0], semis[0])

    @pl.loop(1, NIB, step=2)
    def _(ob):
        for dp in range(2):
            ib = ob + dp
            p = (1 + dp) % 2

            @pl.when(ib < NIB)
            def _():
                pltpu.make_async_copy(
                    src_hbm.at[wid, ib], srcis[p], semis[p]).wait()
                pltpu.make_async_copy(
                    dst_hbm.at[wid, ib], dstis[p], semis[p]).wait()

                do_block(srcis[p], dstis[p], prime=True)

                @pl.when(ib + 2 < NIB)
                def _():
                    pltpu.async_copy(
                        src_hbm.at[wid, ib + 2], srcis[p], semis[p])
                    pltpu.async_copy(
                        dst_hbm.at[wid, ib + 2], dstis[p], semis[p])

    plsc.subcore_barrier()

    out_hbm = [p0_hbm, p1_hbm]
    for c in range(NC):
        @pl.when(cid == c)
        def _():
            @pl.loop(sid, N // ZR, step=NS)
            def _(r):
                ds = pl.ds(r * ZR, ZR)
                pltpu.async_copy(acc.at[ds], out_hbm[c].at[ds], zsem)

            @pl.loop(sid, N // ZR, step=NS)
            def _(r):
                ds = pl.ds(r * ZR, ZR)
                pltpu.make_async_copy(acc.at[ds], out_hbm[c].at[ds],
                                      zsem).wait()


_BLK = 1000


def _root_body(h_ref, wroot_ref, b_ref, o_ref):
    acc = lax.dot_general(
        h_ref[...], wroot_ref[...], (((1,), (0,)), ((), ())),
        precision=lax.Precision.HIGHEST, preferred_element_type=jnp.float32)
    o_ref[...] = acc + b_ref[...]


def _root(h, w_root, b_rel):
    return pl.pallas_call(
        _root_body,
        grid=(N // _BLK,),
        in_specs=[
            pl.BlockSpec((_BLK, D), lambda i: (i, 0)),
            pl.BlockSpec((D, D), lambda i: (0, 0)),
            pl.BlockSpec((1, D), lambda i: (0, 0)),
        ],
        out_specs=pl.BlockSpec((_BLK, D), lambda i: (i, 0)),
        out_shape=jax.ShapeDtypeStruct((N, D), jnp.float32),
    )(h, w_root, b_rel.reshape(1, D))


def _merge_body(do_relu, p0_ref, p1_ref, root_ref, wrel_ref, o_ref):
    s = p0_ref[...] + p1_ref[...]
    acc = lax.dot_general(
        s, wrel_ref[...], (((1,), (0,)), ((), ())),
        precision=lax.Precision.HIGHEST, preferred_element_type=jnp.float32)
    acc = acc + root_ref[...]
    if do_relu:
        acc = jnp.maximum(acc, 0.0)
    o_ref[...] = acc


def _merge(p0, p1, root, w_rel, do_relu):
    return pl.pallas_call(
        functools.partial(_merge_body, do_relu),
        grid=(N // _BLK,),
        in_specs=[
            pl.BlockSpec((_BLK, D), lambda i: (i, 0)),
            pl.BlockSpec((_BLK, D), lambda i: (i, 0)),
            pl.BlockSpec((_BLK, D), lambda i: (i, 0)),
            pl.BlockSpec((D, D), lambda i: (0, 0)),
        ],
        out_specs=pl.BlockSpec((_BLK, D), lambda i: (i, 0)),
        out_shape=jax.ShapeDtypeStruct((N, D), jnp.float32),
    )(p0, p1, root, w_rel)


def kernel(x, edge_index, W_rel0, b_rel0, W_root0, W_rel1, b_rel1, W_root1,
           W_rel2, b_rel2, W_root2):
    src = edge_index[0].reshape(NW, NIB, IBLK, K)
    dst = edge_index[1].reshape(NW, NIB, IBLK, K)
    h = x
    layers = [
        (W_rel0, b_rel0, W_root0, True),
        (W_rel1, b_rel1, W_root1, True),
        (W_rel2, b_rel2, W_root2, False),
    ]
    for w_rel, b_rel, w_root, do_relu in layers:
        p0, p1 = _segsum(h, src, dst)
        root = _root(h, w_root, b_rel)
        h = _merge(p0, p1, root, w_rel, do_relu)
    return h


# speedup vs baseline: 2.2390x; 1.0309x over previous
import dataclasses
import functools

import jax
import jax.numpy as jnp
from jax import lax
from jax.experimental import pallas as pl
from jax.experimental.pallas import tpu as pltpu
from jax.experimental.pallas import tpu_sc as plsc

N = 10000
E = 320000
D = 128

NC = 2
NS = 16
NW = NC * NS

EPW = E // NW
K = 80
NCHUNK = EPW // K
NBUF = 3
ZR = 80
IBLK = 25
NIB = NCHUNK // IBLK

_mesh = plsc.VectorSubcoreMesh(core_axis_name="c", subcore_axis_name="s")

_sc_params = pltpu.CompilerParams()
for _f, _v in (("disable_bounds_checks", True),
               ("disable_semaphore_checks", True)):
    if _f in pltpu.CompilerParams.__dataclass_fields__:
        _sc_params = dataclasses.replace(_sc_params, **{_f: _v})


@functools.partial(
    pl.kernel,
    mesh=_mesh,
    compiler_params=_sc_params,
    out_type=[
        jax.ShapeDtypeStruct((N, D), jnp.float32),
        jax.ShapeDtypeStruct((N, D), jnp.float32),
    ],
    scratch_types=[
        pltpu.VMEM_SHARED((N, D), jnp.float32),
        pltpu.VMEM((IBLK, K), jnp.int32),
        pltpu.VMEM((IBLK, K), jnp.int32),
        pltpu.VMEM((IBLK, K), jnp.int32),
        pltpu.VMEM((IBLK, K), jnp.int32),
        pltpu.VMEM((K, D), jnp.float32),
        pltpu.VMEM((K, D), jnp.float32),
        pltpu.VMEM((K, D), jnp.float32),
        pltpu.SemaphoreType.DMA,
        pltpu.SemaphoreType.DMA,
        pltpu.SemaphoreType.DMA,
        pltpu.SemaphoreType.DMA,
        pltpu.SemaphoreType.DMA,
        pltpu.SemaphoreType.DMA,
        pltpu.SemaphoreType.DMA,
        pltpu.SemaphoreType.DMA,
        pltpu.SemaphoreType.DMA,
    ],
)
def _segsum(h_hbm, src_hbm, dst_hbm, p0_hbm, p1_hbm,
            acc, srci0, srci1, dsti0, dsti1, r0, r1, r2,
            s0, s1, s2, c0, c1, c2, si0, si1, zsem):
    cid = lax.axis_index("c")
    sid = lax.axis_index("s")
    wid = cid * NS + sid
    rows = [r0, r1, r2]
    sems = [s0, s1, s2]
    ssems = [c0, c1, c2]
    srcis = [srci0, srci1]
    dstis = [dsti0, dsti1]
    semis = [si0, si1]
    zbuf = r2

    zeros = jnp.zeros((16,), jnp.float32)

    @pl.loop(0, ZR)
    def _(i):
        @pl.loop(0, D // 16)
        def _(j):
            zbuf[i, pl.ds(j * 16, 16)] = zeros

    @pl.loop(sid, N // ZR, step=NS)
    def _(r):
        pltpu.async_copy(zbuf, acc.at[pl.ds(r * ZR, ZR)], zsem)

    pltpu.async_copy(src_hbm.at[wid, 0], srcis[0], semis[0])
    pltpu.async_copy(dst_hbm.at[wid, 0], dstis[0], semis[0])
    pltpu.async_copy(src_hbm.at[wid, 1], srcis[1], semis[1])
    pltpu.async_copy(dst_hbm.at[wid, 1], dstis[1], semis[1])
    pltpu.make_async_copy(src_hbm.at[wid, 0], srcis[0], semis[0]).wait()
    pltpu.make_async_copy(dst_hbm.at[wid, 0], dstis[0], semis[0]).wait()

    for i in range(NBUF - 1):
        pltpu.async_copy(h_hbm.at[srcis[0].at[i]], rows[i], sems[i])

    @pl.loop(sid, N // ZR, step=NS)
    def _(r):
        pltpu.make_async_copy(zbuf, acc.at[pl.ds(r * ZR, ZR)], zsem).wait()

    plsc.subcore_barrier()

    def do_block(sI, dI, prime):
        if prime:
            for i in range(NBUF - 1):
                pltpu.async_copy(h_hbm.at[sI.at[i]], rows[i], sems[i])

        @pl.loop(0, IBLK - 1, step=NBUF)
        def _(j):
            for b in range(NBUF):
                lc = j + b
                nf = lc + NBUF - 1
                pf = (NBUF - 1 + b) % NBUF

                @pl.when(nf < IBLK)
                def _():
                    def _wait_prev():
                        pltpu.make_async_copy(
                            rows[pf], acc.at[dI.at[lc - 1]],
                            ssems[pf]).wait()
                    if b == 0:
                        pl.when(lc > 0)(_wait_prev)
                    else:
                        _wait_prev()
                    pltpu.async_copy(h_hbm.at[sI.at[nf]], rows[pf], sems[pf])

                pltpu.make_async_copy(
                    h_hbm.at[sI.at[lc]], rows[b], sems[b]).wait()
                pltpu.async_copy(rows[b], acc.at[dI.at[lc]], ssems[b],
                                 add=True)

        pltpu.make_async_copy(
            h_hbm.at[sI.at[IBLK - 1]], rows[0], sems[0]).wait()
        pltpu.async_copy(rows[0], acc.at[dI.at[IBLK - 1]], ssems[0],
                         add=True)
        for c in range(IBLK - NBUF, IBLK):
            s = c % NBUF
            pltpu.make_async_copy(rows[s], acc.at[dI.at[c]], ssems[s]).wait()

    do_block(srcis[0], dstis[0], prime=False)
    pltpu.async_copy(src_hbm.at[wid, 2], srcis[0], semis[0])
    pltpu.async_copy(dst_hbm.at[wid, 2], dstis[0], semis[0])

    @pl.loop(1, NIB, step=2)
    def _(ob):
        for dp in range(2):
            ib = ob + dp
            p = (1 + dp) % 2

            @pl.when(ib < NIB)
            def _():
                pltpu.make_async_copy(
                    src_hbm.at[wid, ib], srcis[p], semis[p]).wait()
                pltpu.make_async_copy(
                    dst_hbm.at[wid, ib], dstis[p], semis[p]).wait()

                do_block(srcis[p], dstis[p], prime=True)

                @pl.when(ib + 2 < NIB)
                def _():
                    pltpu.async_copy(
                        src_hbm.at[wid, ib + 2], srcis[p], semis[p])
                    pltpu.async_copy(
                        dst_hbm.at[wid, ib + 2], dstis[p], semis[p])

    plsc.subcore_barrier()

    out_hbm = [p0_hbm, p1_hbm]
    for c in range(NC):
        @pl.when(cid == c)
        def _():
            @pl.loop(sid, N // ZR, step=NS)
            def _(r):
                ds = pl.ds(r * ZR, ZR)
                pltpu.async_copy(acc.at[ds], out_hbm[c].at[ds], zsem)

            @pl.loop(sid, N // ZR, step=NS)
            def _(r):
                ds = pl.ds(r * ZR, ZR)
                pltpu.make_async_copy(acc.at[ds], out_hbm[c].at[ds],
                                      zsem).wait()


_BLK = 2000


def _root_body(h_ref, wroot_ref, b_ref, o_ref):
    acc = lax.dot_general(
        h_ref[...], wroot_ref[...], (((1,), (0,)), ((), ())),
        precision=lax.Precision.HIGHEST, preferred_element_type=jnp.float32)
    o_ref[...] = acc + b_ref[...]


def _root(h, w_root, b_rel):
    return pl.pallas_call(
        _root_body,
        grid=(N // _BLK,),
        in_specs=[
            pl.BlockSpec((_BLK, D), lambda i: (i, 0)),
            pl.BlockSpec((D, D), lambda i: (0, 0)),
            pl.BlockSpec((1, D), lambda i: (0, 0)),
        ],
        out_specs=pl.BlockSpec((_BLK, D), lambda i: (i, 0)),
        out_shape=jax.ShapeDtypeStruct((N, D), jnp.float32),
    )(h, w_root, b_rel.reshape(1, D))


def _merge_body(do_relu, p0_ref, p1_ref, root_ref, wrel_ref, o_ref):
    s = p0_ref[...] + p1_ref[...]
    acc = lax.dot_general(
        s, wrel_ref[...], (((1,), (0,)), ((), ())),
        precision=lax.Precision.HIGHEST, preferred_element_type=jnp.float32)
    acc = acc + root_ref[...]
    if do_relu:
        acc = jnp.maximum(acc, 0.0)
    o_ref[...] = acc


def _merge(p0, p1, root, w_rel, do_relu):
    return pl.pallas_call(
        functools.partial(_merge_body, do_relu),
        grid=(N // _BLK,),
        in_specs=[
            pl.BlockSpec((_BLK, D), lambda i: (i, 0)),
            pl.BlockSpec((_BLK, D), lambda i: (i, 0)),
            pl.BlockSpec((_BLK, D), lambda i: (i, 0)),
            pl.BlockSpec((D, D), lambda i: (0, 0)),
        ],
        out_specs=pl.BlockSpec((_BLK, D), lambda i: (i, 0)),
        out_shape=jax.ShapeDtypeStruct((N, D), jnp.float32),
    )(p0, p1, root, w_rel)


def kernel(x, edge_index, W_rel0, b_rel0, W_root0, W_rel1, b_rel1, W_root1,
           W_rel2, b_rel2, W_root2):
    src = edge_index[0].reshape(NW, NIB, IBLK, K)
    dst = edge_index[1].reshape(NW, NIB, IBLK, K)
    h = x
    layers = [
        (W_rel0, b_rel0, W_root0, True),
        (W_rel1, b_rel1, W_root1, True),
        (W_rel2, b_rel2, W_root2, False),
    ]
    for w_rel, b_rel, w_root, do_relu in layers:
        p0, p1 = _segsum(h, src, dst)
        root = _root(h, w_root, b_rel)
        h = _merge(p0, p1, root, w_rel, do_relu)
    return h
